# Initial kernel scaffold; baseline (speedup 1.0000x reference)
import jax
import jax.numpy as jnp
from jax.experimental import pallas as pl


def kernel(node0, node1, adj0, adj1, mask, feat_g0_l0_w, feat_g0_l0_b, feat_g0_l1_w, feat_g0_l1_b, feat_g1_l0_w, feat_g1_l0_b, feat_g1_l1_w, feat_g1_l1_b, msg_g0_l0_w, msg_g0_l0_b, msg_g0_l1_w, msg_g0_l1_b, msg_g1_l0_w, msg_g1_l0_b, msg_g1_l1_w, msg_g1_l1_b, merge_l0_w, merge_l0_b, merge_l1_w, merge_l1_b, fout_l0_w, fout_l0_b, fout_l1_w, fout_l1_b, prio_l0_w, prio_l0_b, prio_l1_w, prio_l1_b, aggp_l0_w, aggp_l0_b, aggp_l1_w, aggp_l1_b):
    raise NotImplementedError("write your pallas kernel here")


# speedup vs baseline: 1.7744x; 1.7744x over previous
import math

import jax
import jax.numpy as jnp
from jax.experimental import pallas as pl
from jax.experimental.pallas import tpu as pltpu

LEAKY_SLOPE = 0.01
LOG_MASK_EPS = math.log(1e-45)
N_STEPS = 4


def _leaky(x):
    return jnp.maximum(x, LEAKY_SLOPE * x)


def _mlp2(x_bf16, w0, b0, w1, b1, act_last=True):
    y = jnp.dot(x_bf16, w0, preferred_element_type=jnp.float32) + b0
    y = _leaky(y)
    y = jnp.dot(y.astype(jnp.bfloat16), w1,
                preferred_element_type=jnp.float32) + b1
    if act_last:
        y = _leaky(y)
    return y


def kernel(node0, node1, adj0, adj1, mask,
           feat_g0_l0_w, feat_g0_l0_b, feat_g0_l1_w, feat_g0_l1_b,
           feat_g1_l0_w, feat_g1_l0_b, feat_g1_l1_w, feat_g1_l1_b,
           msg_g0_l0_w, msg_g0_l0_b, msg_g0_l1_w, msg_g0_l1_b,
           msg_g1_l0_w, msg_g1_l0_b, msg_g1_l1_w, msg_g1_l1_b,
           merge_l0_w, merge_l0_b, merge_l1_w, merge_l1_b,
           fout_l0_w, fout_l0_b, fout_l1_w, fout_l1_b,
           prio_l0_w, prio_l0_b, prio_l1_w, prio_l1_b,
           aggp_l0_w, aggp_l0_b, aggp_l1_w, aggp_l1_b):
    B, N, F0 = node0.shape
    F1 = node1.shape[2]
    S1 = mask.shape[1]
    S = S1 - 1
    HS = feat_g0_l1_w.shape[1]
    NOUT = merge_l1_w.shape[1]

    Bt = B // 2 if B % 2 == 0 else B
    n_blocks = B // Bt

    bf = jnp.bfloat16

    def row(b):
        return b.reshape(1, -1)

    ins = [
        node0, node1, adj0, adj1, mask.astype(jnp.float32),
        feat_g0_l0_w.astype(bf), row(feat_g0_l0_b),
        feat_g0_l1_w.astype(bf), row(feat_g0_l1_b),
        feat_g1_l0_w.astype(bf), row(feat_g1_l0_b),
        feat_g1_l1_w.astype(bf), row(feat_g1_l1_b),
        msg_g0_l0_w.astype(bf), row(msg_g0_l0_b),
        msg_g0_l1_w.astype(bf), row(msg_g0_l1_b),
        msg_g1_l0_w.astype(bf), row(msg_g1_l0_b),
        msg_g1_l1_w.astype(bf), row(msg_g1_l1_b),
        merge_l0_w.astype(bf), row(merge_l0_b),
        merge_l1_w.astype(bf), row(merge_l1_b),
        fout_l0_w.astype(bf), row(fout_l0_b),
        fout_l1_w.astype(bf), row(fout_l1_b),
        prio_l0_w.astype(bf), row(prio_l0_b),
        prio_l1_w, row(prio_l1_b),
        aggp_l0_w, row(aggp_l0_b),
        aggp_l1_w, row(aggp_l1_b),
    ]

    def body(n0, n1, a0, a1, msk,
             wf00, bf00, wf01, bf01, wf10, bf10, wf11, bf11,
             wm00, bm00, wm01, bm01, wm10, bm10, wm11, bm11,
             wg0, bg0, wg1, bg1,
             wo0, bo0, wo1, bo1,
             wp0, bp0, wp1, bp1,
             wa0, ba0, wa1, ba1,
             out_lp, out_p, out_mlp, out_mp):
        x0 = n0[...].astype(bf).reshape(Bt * N, F0)
        x1 = n1[...].astype(bf).reshape(Bt * N, F1)
        h0 = _mlp2(x0, wf00[...], bf00[...], wf01[...], bf01[...])
        h1 = _mlp2(x1, wf10[...], bf10[...], wf11[...], bf11[...])
        adj0b = a0[...].astype(bf)
        adj1b = a1[...].astype(bf)

        dn = (((2,), (1,)), ((0,), (0,)))
        for _ in range(N_STEPS):
            m0 = _mlp2(h0.astype(bf), wm00[...], bm00[...],
                       wm01[...], bm01[...]).astype(bf)
            m1 = _mlp2(h1.astype(bf), wm10[...], bm10[...],
                       wm11[...], bm11[...]).astype(bf)
            d0 = jax.lax.dot_general(adj0b, m0.reshape(Bt, N, HS), dn,
                                     preferred_element_type=jnp.float32)
            d1 = jax.lax.dot_general(adj1b, m1.reshape(Bt, N, HS), dn,
                                     preferred_element_type=jnp.float32)
            h0 = h0 + d0.reshape(Bt * N, HS)
            h1 = h1 + d1.reshape(Bt * N, HS)

        hcat = jnp.concatenate([h0, h1], axis=-1).astype(bf)
        gcn = _mlp2(hcat, wg0[...], bg0[...], wg1[...], bg1[...])

        sw_in = gcn.reshape(Bt, N, NOUT)[:, :S, :].reshape(Bt * S, NOUT)
        sw = _mlp2(sw_in.astype(bf), wo0[...], bo0[...], wo1[...], bo1[...])

        p1 = _leaky(jnp.dot(sw.astype(bf), wp0[...],
                            preferred_element_type=jnp.float32) + bp0[...])
        sp = jnp.dot(p1, wp1[...],
                     preferred_element_type=jnp.float32) + bp1[...]

        agg = jnp.sum(sw.reshape(Bt, S, NOUT), axis=1)
        a1h = _leaky(jnp.dot(agg, wa0[...],
                             preferred_element_type=jnp.float32) + ba0[...])
        tp = jnp.dot(a1h, wa1[...],
                     preferred_element_type=jnp.float32) + ba1[...]

        sp2 = sp.reshape(Bt, S)
        pv = jnp.concatenate([sp2, tp], axis=1)

        m = jnp.max(pv, axis=1, keepdims=True)
        z = pv - m
        lse = jnp.log(jnp.sum(jnp.exp(z), axis=1, keepdims=True))
        log_pi = z - lse

        mv = msk[...]
        log_mask = jnp.where(mv > 0.5, jnp.float32(0.0),
                             jnp.float32(LOG_MASK_EPS))
        pvm = pv + log_mask
        m2 = jnp.max(pvm, axis=1, keepdims=True)
        z2 = pvm - m2
        lse2 = jnp.log(jnp.sum(jnp.exp(z2), axis=1, keepdims=True))
        mlog_pi = z2 - lse2

        out_lp[...] = log_pi
        out_p[...] = jnp.exp(log_pi)
        out_mlp[...] = mlog_pi
        out_mp[...] = jnp.exp(mlog_pi)

    def bspec(shape):
        nd = len(shape)
        return pl.BlockSpec((Bt,) + shape[1:],
                            lambda i, nd=nd: (i,) + (0,) * (nd - 1))

    def wspec(shape):
        nd = len(shape)
        return pl.BlockSpec(shape, lambda i, nd=nd: (0,) * nd)

    in_specs = [bspec(node0.shape), bspec(node1.shape),
                bspec(adj0.shape), bspec(adj1.shape),
                bspec((B, S1))]
    in_specs += [wspec(a.shape) for a in ins[5:]]

    out_specs = [pl.BlockSpec((Bt, S1), lambda i: (i, 0))] * 4
    out_shape = [jax.ShapeDtypeStruct((B, S1), jnp.float32)] * 4

    outs = pl.pallas_call(
        body,
        grid=(n_blocks,),
        in_specs=in_specs,
        out_specs=out_specs,
        out_shape=out_shape,
        compiler_params=pltpu.CompilerParams(
            dimension_semantics=("parallel",)),
    )(*ins)
    return tuple(outs)


# speedup vs baseline: 2.9443x; 1.6593x over previous
import math

import jax
import jax.numpy as jnp
from jax.experimental import pallas as pl
from jax.experimental.pallas import tpu as pltpu

LEAKY_SLOPE = 0.01
LOG_MASK_EPS = math.log(1e-45)
N_STEPS = 4


def _leaky(x):
    return jnp.maximum(x, LEAKY_SLOPE * x)


def _mlp2(x_bf16, w0, b0, w1, b1, act_last=True):
    y = jnp.dot(x_bf16, w0, preferred_element_type=jnp.float32) + b0
    y = _leaky(y)
    y = jnp.dot(y.astype(jnp.bfloat16), w1,
                preferred_element_type=jnp.float32) + b1
    if act_last:
        y = _leaky(y)
    return y


def kernel(node0, node1, adj0, adj1, mask,
           feat_g0_l0_w, feat_g0_l0_b, feat_g0_l1_w, feat_g0_l1_b,
           feat_g1_l0_w, feat_g1_l0_b, feat_g1_l1_w, feat_g1_l1_b,
           msg_g0_l0_w, msg_g0_l0_b, msg_g0_l1_w, msg_g0_l1_b,
           msg_g1_l0_w, msg_g1_l0_b, msg_g1_l1_w, msg_g1_l1_b,
           merge_l0_w, merge_l0_b, merge_l1_w, merge_l1_b,
           fout_l0_w, fout_l0_b, fout_l1_w, fout_l1_b,
           prio_l0_w, prio_l0_b, prio_l1_w, prio_l1_b,
           aggp_l0_w, aggp_l0_b, aggp_l1_w, aggp_l1_b):
    B, N, F0 = node0.shape
    F1 = node1.shape[2]
    S1 = mask.shape[1]
    S = S1 - 1
    HS = feat_g0_l1_w.shape[1]
    NOUT = merge_l1_w.shape[1]

    Bt = B // 2 if B % 2 == 0 else B
    n_blocks = B // Bt

    bf = jnp.bfloat16

    def row(b):
        return b.reshape(1, -1)

    ins = [
        node0, node1, adj0, adj1, mask,
        feat_g0_l0_w, row(feat_g0_l0_b),
        feat_g0_l1_w, row(feat_g0_l1_b),
        feat_g1_l0_w, row(feat_g1_l0_b),
        feat_g1_l1_w, row(feat_g1_l1_b),
        msg_g0_l0_w, row(msg_g0_l0_b),
        msg_g0_l1_w, row(msg_g0_l1_b),
        msg_g1_l0_w, row(msg_g1_l0_b),
        msg_g1_l1_w, row(msg_g1_l1_b),
        merge_l0_w, row(merge_l0_b),
        merge_l1_w, row(merge_l1_b),
        fout_l0_w, row(fout_l0_b),
        fout_l1_w, row(fout_l1_b),
        prio_l0_w, row(prio_l0_b),
        prio_l1_w, row(prio_l1_b),
        aggp_l0_w, row(aggp_l0_b),
        aggp_l1_w, row(aggp_l1_b),
    ]

    def body(n0, n1, a0, a1, msk,
             wf00, bf00, wf01, bf01, wf10, bf10, wf11, bf11,
             wm00, bm00, wm01, bm01, wm10, bm10, wm11, bm11,
             wg0, bg0, wg1, bg1,
             wo0, bo0, wo1, bo1,
             wp0, bp0, wp1, bp1,
             wa0, ba0, wa1, ba1,
             out_lp, out_p, out_mlp, out_mp):
        x0 = n0[...].astype(bf).reshape(Bt * N, F0)
        x1 = n1[...].astype(bf).reshape(Bt * N, F1)
        h0 = _mlp2(x0, wf00[...].astype(bf), bf00[...],
                   wf01[...].astype(bf), bf01[...])
        h1 = _mlp2(x1, wf10[...].astype(bf), bf10[...],
                   wf11[...].astype(bf), bf11[...])
        adj0b = a0[...].astype(bf)
        adj1b = a1[...].astype(bf)

        dn = (((2,), (1,)), ((0,), (0,)))
        for _ in range(N_STEPS):
            m0 = _mlp2(h0.astype(bf), wm00[...].astype(bf), bm00[...],
                       wm01[...].astype(bf), bm01[...]).astype(bf)
            m1 = _mlp2(h1.astype(bf), wm10[...].astype(bf), bm10[...],
                       wm11[...].astype(bf), bm11[...]).astype(bf)
            d0 = jax.lax.dot_general(adj0b, m0.reshape(Bt, N, HS), dn,
                                     preferred_element_type=jnp.float32)
            d1 = jax.lax.dot_general(adj1b, m1.reshape(Bt, N, HS), dn,
                                     preferred_element_type=jnp.float32)
            h0 = h0 + d0.reshape(Bt * N, HS)
            h1 = h1 + d1.reshape(Bt * N, HS)

        hcat = jnp.concatenate([h0, h1], axis=-1).astype(bf)
        gcn = _mlp2(hcat, wg0[...].astype(bf), bg0[...],
                    wg1[...].astype(bf), bg1[...])

        sw_in = gcn.reshape(Bt, N, NOUT)[:, :S, :].reshape(Bt * S, NOUT)
        sw = _mlp2(sw_in.astype(bf), wo0[...].astype(bf), bo0[...],
                   wo1[...].astype(bf), bo1[...])

        p1 = _leaky(jnp.dot(sw.astype(bf), wp0[...].astype(bf),
                            preferred_element_type=jnp.float32) + bp0[...])
        sp = jnp.dot(p1, wp1[...],
                     preferred_element_type=jnp.float32) + bp1[...]

        agg = jnp.sum(sw.reshape(Bt, S, NOUT), axis=1)
        a1h = _leaky(jnp.dot(agg, wa0[...],
                             preferred_element_type=jnp.float32) + ba0[...])
        tp = jnp.dot(a1h, wa1[...],
                     preferred_element_type=jnp.float32) + ba1[...]

        sp2 = sp.reshape(Bt, S)
        pv = jnp.concatenate([sp2, tp], axis=1)

        m = jnp.max(pv, axis=1, keepdims=True)
        z = pv - m
        lse = jnp.log(jnp.sum(jnp.exp(z), axis=1, keepdims=True))
        log_pi = z - lse

        mv = msk[...]
        log_mask = jnp.where(mv > 0.5, jnp.float32(0.0),
                             jnp.float32(LOG_MASK_EPS))
        pvm = pv + log_mask
        m2 = jnp.max(pvm, axis=1, keepdims=True)
        z2 = pvm - m2
        lse2 = jnp.log(jnp.sum(jnp.exp(z2), axis=1, keepdims=True))
        mlog_pi = z2 - lse2

        out_lp[...] = log_pi
        out_p[...] = jnp.exp(log_pi)
        out_mlp[...] = mlog_pi
        out_mp[...] = jnp.exp(mlog_pi)

    def bspec(shape):
        nd = len(shape)
        return pl.BlockSpec((Bt,) + shape[1:],
                            lambda i, nd=nd: (i,) + (0,) * (nd - 1))

    def wspec(shape):
        nd = len(shape)
        return pl.BlockSpec(shape, lambda i, nd=nd: (0,) * nd)

    in_specs = [bspec(node0.shape), bspec(node1.shape),
                bspec(adj0.shape), bspec(adj1.shape),
                bspec((B, S1))]
    in_specs += [wspec(a.shape) for a in ins[5:]]

    out_specs = [pl.BlockSpec((Bt, S1), lambda i: (i, 0))] * 4
    out_shape = [jax.ShapeDtypeStruct((B, S1), jnp.float32)] * 4

    outs = pl.pallas_call(
        body,
        grid=(n_blocks,),
        in_specs=in_specs,
        out_specs=out_specs,
        out_shape=out_shape,
        compiler_params=pltpu.CompilerParams(
            dimension_semantics=("parallel",)),
    )(*ins)
    return tuple(outs)
